# baseline (speedup 1.0000x reference)
import functools

import jax
import jax.numpy as jnp
from jax import lax
from jax.experimental import pallas as pl
from jax.experimental.pallas import tpu as pltpu
from jax.experimental.pallas import tpu_sc as plsc

D_MODEL_K = 1024
SCALE = float(D_MODEL_K) ** 0.5

NW = 32
B_TOTAL = 4 * 4096
B_PER_W = B_TOTAL // NW
CHUNK = 24
NFULL = B_PER_W // CHUNK
TAIL = B_PER_W - NFULL * CHUNK
NCHUNK = NFULL + 1
NBUF = 4
LANES = 16
VPR = D_MODEL_K // LANES


def _chunk_rows(g):
    return CHUNK if g < NFULL else TAIL


@functools.partial(
    pl.kernel,
    out_type=jax.ShapeDtypeStruct((B_TOTAL, D_MODEL_K), jnp.float32),
    mesh=plsc.VectorSubcoreMesh(core_axis_name="c", subcore_axis_name="s"),
    scratch_types=(
        [pltpu.VMEM((B_PER_W,), jnp.int32)]
        + [pltpu.VMEM((CHUNK, D_MODEL_K), jnp.float32) for _ in range(NBUF)]
        + [pltpu.SemaphoreType.DMA for _ in range(2 * NBUF)]
    ),
)
def _emb_lookup(x_hbm, table_hbm, out_hbm, idx_v, *bufs_and_sems):
    bufs = bufs_and_sems[:NBUF]
    gsem = bufs_and_sems[NBUF:2 * NBUF]
    ssem = bufs_and_sems[2 * NBUF:]
    cid = lax.axis_index("c")
    sid = lax.axis_index("s")
    wid = sid * 2 + cid
    base = wid * B_PER_W
    pltpu.sync_copy(x_hbm.at[wid], idx_v)

    def start_gather(g):
        b = g % NBUF
        n = _chunk_rows(g)
        dst = bufs[b] if n == CHUNK else bufs[b].at[pl.ds(0, TAIL)]
        return pltpu.async_copy(
            table_hbm.at[idx_v.at[pl.ds(g * CHUNK, n)]], dst, gsem[b])

    def scale_buf(b, nrows):
        rows = bufs[b]

        def scale_quarter_row(i, _):
            r = lax.shift_right_logical(i, 2)
            h = lax.bitwise_and(i, 3) * (VPR // 4)
            for j in range(VPR // 4):
                sl = pl.ds((h + j) * LANES, LANES)
                rows[r, sl] = rows[r, sl] * SCALE
            return 0

        lax.fori_loop(0, nrows * 4, scale_quarter_row, 0)

    def start_store(g):
        b = g % NBUF
        n = _chunk_rows(g)
        src = bufs[b] if n == CHUNK else bufs[b].at[pl.ds(0, TAIL)]
        return pltpu.async_copy(
            src, out_hbm.at[pl.ds(base + g * CHUNK, n)], ssem[b])

    gh, sh = {}, {}
    gh[0] = start_gather(0)
    for g in range(NCHUNK):
        b = g % NBUF
        for a in (1, 2):
            if g + a < NCHUNK and (g + a) not in gh:
                d = g + a - NBUF
                if d in sh:
                    sh.pop(d).wait()
                gh[g + a] = start_gather(g + a)
        gh.pop(g).wait()
        scale_buf(b, _chunk_rows(g))
        sh[g] = start_store(g)
    for g in sorted(sh):
        sh.pop(g).wait()


def kernel(x, table):
    xr = x.reshape(NW, B_PER_W)
    out = _emb_lookup(xr, table)
    return out.reshape(4, 4096, D_MODEL_K)


# speedup vs baseline: 1.0046x; 1.0046x over previous
import functools

import jax
import jax.numpy as jnp
from jax import lax
from jax.experimental import pallas as pl
from jax.experimental.pallas import tpu as pltpu
from jax.experimental.pallas import tpu_sc as plsc

D_MODEL_K = 1024
SCALE = float(D_MODEL_K) ** 0.5

NW = 32
B_TOTAL = 4 * 4096
B_PER_W = B_TOTAL // NW
CHUNK = 40
NFULL = B_PER_W // CHUNK
TAIL = B_PER_W - NFULL * CHUNK
NCHUNK = NFULL + 1
NBUF = 3
LANES = 16
VPR = D_MODEL_K // LANES


def _chunk_rows(g):
    return CHUNK if g < NFULL else TAIL


@functools.partial(
    pl.kernel,
    out_type=jax.ShapeDtypeStruct((B_TOTAL, D_MODEL_K), jnp.float32),
    mesh=plsc.VectorSubcoreMesh(core_axis_name="c", subcore_axis_name="s"),
    scratch_types=(
        [pltpu.VMEM((B_PER_W,), jnp.int32)]
        + [pltpu.VMEM((CHUNK, D_MODEL_K), jnp.float32) for _ in range(NBUF)]
        + [pltpu.SemaphoreType.DMA for _ in range(2 * NBUF)]
    ),
)
def _emb_lookup(x_hbm, table_hbm, out_hbm, idx_v, *bufs_and_sems):
    bufs = bufs_and_sems[:NBUF]
    gsem = bufs_and_sems[NBUF:2 * NBUF]
    ssem = bufs_and_sems[2 * NBUF:]
    cid = lax.axis_index("c")
    sid = lax.axis_index("s")
    wid = sid * 2 + cid
    base = wid * B_PER_W
    pltpu.sync_copy(x_hbm.at[wid], idx_v)

    def start_gather(g):
        b = g % NBUF
        n = _chunk_rows(g)
        dst = bufs[b] if n == CHUNK else bufs[b].at[pl.ds(0, TAIL)]
        return pltpu.async_copy(
            table_hbm.at[idx_v.at[pl.ds(g * CHUNK, n)]], dst, gsem[b])

    def scale_buf(b, nrows):
        rows = bufs[b]

        def scale_quarter_row(i, _):
            r = lax.shift_right_logical(i, 2)
            h = lax.bitwise_and(i, 3) * (VPR // 4)
            for j in range(VPR // 4):
                sl = pl.ds((h + j) * LANES, LANES)
                rows[r, sl] = rows[r, sl] * SCALE
            return 0

        lax.fori_loop(0, nrows * 4, scale_quarter_row, 0)

    def start_store(g):
        b = g % NBUF
        n = _chunk_rows(g)
        src = bufs[b] if n == CHUNK else bufs[b].at[pl.ds(0, TAIL)]
        return pltpu.async_copy(
            src, out_hbm.at[pl.ds(base + g * CHUNK, n)], ssem[b])

    gh, sh = {}, {}
    gh[0] = start_gather(0)
    for g in range(NCHUNK):
        b = g % NBUF
        if g + 1 < NCHUNK:
            if g - 2 >= 0:
                sh.pop(g - 2).wait()
            gh[g + 1] = start_gather(g + 1)
        gh.pop(g).wait()
        scale_buf(b, _chunk_rows(g))
        sh[g] = start_store(g)
    for g in sorted(sh):
        sh.pop(g).wait()


def kernel(x, table):
    xr = x.reshape(NW, B_PER_W)
    out = _emb_lookup(xr, table)
    return out.reshape(4, 4096, D_MODEL_K)
